# Initial kernel scaffold; baseline (speedup 1.0000x reference)
import functools

import jax
import jax.numpy as jnp
from jax import lax
from jax.experimental import pallas as pl
from jax.experimental.pallas import tpu as pltpu
from jax.experimental.pallas import tpu_sc as plsc

N_NODES = 10000
N_HEDGES = 5000
N_PAIRS = 320000
D = 128

NC = 2
NS = 16
NW = NC * NS

CHUNK = 128
VP = 327680
CPT = VP // (NW * CHUNK)
EPAD = 5120
NPAD = 10240
WIDTH = 144
E_ROWS_PER_TILE = EPAD // NS
N_ROWS_PER_TILE = NPAD // NS


def _sc_mesh():
    return plsc.VectorSubcoreMesh(core_axis_name="c", subcore_axis_name="s")


def _edge_accum_body(xaug, vidx, eidx, zrows, out, idxv, idxe, buf, acc, sem):
    c = lax.axis_index("c")
    s = lax.axis_index("s")
    wid = c * NS + s
    pltpu.sync_copy(zrows, acc.at[pl.ds(s * E_ROWS_PER_TILE, E_ROWS_PER_TILE)])
    pltpu.sync_copy(vidx.at[wid], idxv)
    pltpu.sync_copy(eidx.at[wid], idxe)
    plsc.subcore_barrier()

    def body(j, carry):
        pltpu.async_copy(xaug.at[idxv.at[j]], buf, sem).wait()
        pltpu.sync_copy(buf, acc.at[idxe.at[j]], add=True)
        return carry

    lax.fori_loop(0, CPT, body, 0)
    plsc.subcore_barrier()
    pltpu.sync_copy(acc.at[pl.ds(s * E_ROWS_PER_TILE, E_ROWS_PER_TILE)],
                    out.at[c, pl.ds(s * E_ROWS_PER_TILE, E_ROWS_PER_TILE)])


def _node_accum_body(xe, eidx, vidx, zrows, out, idxe, idxv, buf, acc, sem):
    c = lax.axis_index("c")
    s = lax.axis_index("s")
    wid = c * NS + s
    pltpu.sync_copy(zrows, acc.at[pl.ds(s * N_ROWS_PER_TILE, N_ROWS_PER_TILE)])
    pltpu.sync_copy(eidx.at[wid], idxe)
    pltpu.sync_copy(vidx.at[wid], idxv)
    plsc.subcore_barrier()

    def body(j, carry):
        pltpu.async_copy(xe.at[idxe.at[j]], buf, sem).wait()
        pltpu.sync_copy(buf, acc.at[idxv.at[j]], add=True)
        return carry

    lax.fori_loop(0, CPT, body, 0)
    plsc.subcore_barrier()
    pltpu.sync_copy(acc.at[pl.ds(s * N_ROWS_PER_TILE, N_ROWS_PER_TILE)],
                    out.at[c, pl.ds(s * N_ROWS_PER_TILE, N_ROWS_PER_TILE)])


def _edge_mean_body(esum_ref, xe_ref):
    s = esum_ref[0] + esum_ref[1]
    feat = s[:, :D]
    cnt = s[:, D:D + 1]
    xe_ref[...] = feat / jnp.maximum(cnt, 1.0)


def _project_body(x_ref, xv_ref, w_ref, scale_ref, o_ref):
    xn = scale_ref[0, 0] * x_ref[...] + xv_ref[0] + xv_ref[1]
    o_ref[...] = lax.dot_general(
        xn, w_ref[...], (((1,), (1,)), ((), ())),
        preferred_element_type=jnp.float32)


def kernel(X, vertex, edges, W, eps):
    pad = VP - N_PAIRS
    xaug = jnp.concatenate(
        [X, jnp.ones((N_NODES, 1), jnp.float32),
         jnp.zeros((N_NODES, WIDTH - D - 1), jnp.float32)], axis=1)
    v_gather = jnp.concatenate(
        [vertex, jnp.zeros((pad,), jnp.int32)]).reshape(NW, CPT, CHUNK)
    v_scatter = jnp.concatenate(
        [vertex, jnp.full((pad,), N_NODES, jnp.int32)]).reshape(NW, CPT, CHUNK)
    e_idx = jnp.concatenate(
        [edges, jnp.full((pad,), N_HEDGES, jnp.int32)]).reshape(NW, CPT, CHUNK)
    z_edge = jnp.zeros((E_ROWS_PER_TILE, WIDTH), jnp.float32)
    z_node = jnp.zeros((N_ROWS_PER_TILE, D), jnp.float32)

    ka = pl.kernel(
        _edge_accum_body,
        out_type=jax.ShapeDtypeStruct((NC, EPAD, WIDTH), jnp.float32),
        mesh=_sc_mesh(),
        scratch_types=[
            pltpu.VMEM((CPT, CHUNK), jnp.int32),
            pltpu.VMEM((CPT, CHUNK), jnp.int32),
            pltpu.VMEM((CHUNK, WIDTH), jnp.float32),
            pltpu.VMEM_SHARED((EPAD, WIDTH), jnp.float32),
            pltpu.SemaphoreType.DMA,
        ],
    )
    esum = ka(xaug, v_gather, e_idx, z_edge)

    xe = pl.pallas_call(
        _edge_mean_body,
        out_shape=jax.ShapeDtypeStruct((EPAD, D), jnp.float32),
    )(esum)

    kc = pl.kernel(
        _node_accum_body,
        out_type=jax.ShapeDtypeStruct((NC, NPAD, D), jnp.float32),
        mesh=_sc_mesh(),
        scratch_types=[
            pltpu.VMEM((CPT, CHUNK), jnp.int32),
            pltpu.VMEM((CPT, CHUNK), jnp.int32),
            pltpu.VMEM((CHUNK, D), jnp.float32),
            pltpu.VMEM_SHARED((NPAD, D), jnp.float32),
            pltpu.SemaphoreType.DMA,
        ],
    )
    xv = kc(xe, e_idx, v_scatter, z_node)

    scale = jnp.reshape(1.0 + eps[0], (1, 1))
    rows_blk = 1000
    out = pl.pallas_call(
        _project_body,
        grid=(N_NODES // rows_blk,),
        in_specs=[
            pl.BlockSpec((rows_blk, D), lambda i: (i, 0)),
            pl.BlockSpec((NC, rows_blk, D), lambda i: (0, i, 0)),
            pl.BlockSpec((D, D), lambda i: (0, 0)),
            pl.BlockSpec(memory_space=pltpu.SMEM),
        ],
        out_specs=pl.BlockSpec((rows_blk, D), lambda i: (i, 0)),
        out_shape=jax.ShapeDtypeStruct((N_NODES, D), jnp.float32),
    )(X, xv, W, scale)
    return out


# speedup vs baseline: 2.8018x; 2.8018x over previous
import jax
import jax.numpy as jnp
from jax import lax
from jax.experimental import pallas as pl
from jax.experimental.pallas import tpu as pltpu
from jax.experimental.pallas import tpu_sc as plsc

N_NODES = 10000
N_HEDGES = 5000
N_PAIRS = 320000
D = 128

NC = 2
NS = 16
NW = NC * NS
L = 16

CHUNK = 128
VP = 327680
CPT = VP // (NW * CHUNK)
EPAD = 5120
NPAD = 10240
E_ROWS_PER_TILE = EPAD // NS
N_ROWS_PER_TILE = NPAD // NS


def _sc_mesh():
    return plsc.VectorSubcoreMesh(core_axis_name="c", subcore_axis_name="s")


def _edge_accum_body(x, vidx, eidx, z_edge, out, out_cnt,
                     idxv, idxe, buf, ones, znb, cntb, acc, cnt, sem):
    c = lax.axis_index("c")
    s = lax.axis_index("s")
    wid = c * NS + s
    pltpu.sync_copy(z_edge, acc.at[pl.ds(s * E_ROWS_PER_TILE, E_ROWS_PER_TILE)])

    def zbody(g, h):
        znb[pl.ds(g * L, L)] = jnp.zeros((L,), jnp.float32)
        return h
    lax.fori_loop(0, E_ROWS_PER_TILE // L, zbody, 0)
    pltpu.sync_copy(znb, cnt.at[pl.ds(s * E_ROWS_PER_TILE, E_ROWS_PER_TILE)])

    def obody(g, h):
        ones[pl.ds(g * L, L)] = jnp.ones((L,), jnp.float32)
        return h
    lax.fori_loop(0, CHUNK // L, obody, 0)

    pltpu.sync_copy(vidx.at[wid], idxv)
    pltpu.sync_copy(eidx.at[wid], idxe)
    plsc.subcore_barrier()

    def body(j, carry):
        pltpu.async_copy(x.at[idxv.at[j]], buf, sem).wait()
        pltpu.sync_copy(buf, acc.at[idxe.at[j]], add=True)
        pltpu.sync_copy(ones, cnt.at[idxe.at[j]], add=True)
        return carry

    lax.fori_loop(0, CPT, body, 0)
    plsc.subcore_barrier()
    pltpu.sync_copy(acc.at[pl.ds(s * E_ROWS_PER_TILE, E_ROWS_PER_TILE)],
                    out.at[c, pl.ds(s * E_ROWS_PER_TILE, E_ROWS_PER_TILE)])

    @pl.when(s == 0)
    def _():
        pltpu.sync_copy(cnt, cntb)
        pltpu.sync_copy(cntb, out_cnt.at[c, 0])


def _node_accum_body(xe, eidx, vidx, z_node, out, idxe, idxv, buf, acc, sem):
    c = lax.axis_index("c")
    s = lax.axis_index("s")
    wid = c * NS + s
    pltpu.sync_copy(z_node, acc.at[pl.ds(s * N_ROWS_PER_TILE, N_ROWS_PER_TILE)])
    pltpu.sync_copy(eidx.at[wid], idxe)
    pltpu.sync_copy(vidx.at[wid], idxv)
    plsc.subcore_barrier()

    def body(j, carry):
        pltpu.async_copy(xe.at[idxe.at[j]], buf, sem).wait()
        pltpu.sync_copy(buf, acc.at[idxv.at[j]], add=True)
        return carry

    lax.fori_loop(0, CPT, body, 0)
    plsc.subcore_barrier()
    pltpu.sync_copy(acc.at[pl.ds(s * N_ROWS_PER_TILE, N_ROWS_PER_TILE)],
                    out.at[c, pl.ds(s * N_ROWS_PER_TILE, N_ROWS_PER_TILE)])


def _edge_mean_body(esum_ref, cnt_ref, xe_ref):
    s = esum_ref[0] + esum_ref[1]
    cnt = cnt_ref[0] + cnt_ref[1]
    xe_ref[...] = s / jnp.maximum(cnt, 1.0)[:, None]


def _project_body(x_ref, xv_ref, w_ref, scale_ref, o_ref):
    xn = scale_ref[0, 0] * x_ref[...] + xv_ref[0] + xv_ref[1]
    o_ref[...] = lax.dot_general(
        xn, w_ref[...], (((1,), (1,)), ((), ())),
        preferred_element_type=jnp.float32)


def kernel(X, vertex, edges, W, eps):
    pad = VP - N_PAIRS
    v_gather = jnp.concatenate(
        [vertex, jnp.zeros((pad,), jnp.int32)]).reshape(NW, CPT, CHUNK)
    v_scatter = jnp.concatenate(
        [vertex, jnp.full((pad,), N_NODES, jnp.int32)]).reshape(NW, CPT, CHUNK)
    e_idx = jnp.concatenate(
        [edges, jnp.full((pad,), N_HEDGES, jnp.int32)]).reshape(NW, CPT, CHUNK)
    z_edge = jnp.zeros((E_ROWS_PER_TILE, D), jnp.float32)
    z_node = jnp.zeros((N_ROWS_PER_TILE, D), jnp.float32)

    ka = pl.kernel(
        _edge_accum_body,
        out_type=(jax.ShapeDtypeStruct((NC, EPAD, D), jnp.float32),
                  jax.ShapeDtypeStruct((NC, 1, EPAD), jnp.float32)),
        mesh=_sc_mesh(),
        scratch_types=[
            pltpu.VMEM((CPT, CHUNK), jnp.int32),
            pltpu.VMEM((CPT, CHUNK), jnp.int32),
            pltpu.VMEM((CHUNK, D), jnp.float32),
            pltpu.VMEM((CHUNK,), jnp.float32),
            pltpu.VMEM((E_ROWS_PER_TILE,), jnp.float32),
            pltpu.VMEM((EPAD,), jnp.float32),
            pltpu.VMEM_SHARED((EPAD, D), jnp.float32),
            pltpu.VMEM_SHARED((EPAD,), jnp.float32),
            pltpu.SemaphoreType.DMA,
        ],
    )
    esum, ecnt = ka(X, v_gather, e_idx, z_edge)

    blk = 640
    xe = pl.pallas_call(
        _edge_mean_body,
        grid=(EPAD // blk,),
        in_specs=[
            pl.BlockSpec((NC, blk, D), lambda i: (0, i, 0)),
            pl.BlockSpec((NC, blk), lambda i: (0, i)),
        ],
        out_specs=pl.BlockSpec((blk, D), lambda i: (i, 0)),
        out_shape=jax.ShapeDtypeStruct((EPAD, D), jnp.float32),
    )(esum, ecnt.reshape(NC, EPAD))

    kc = pl.kernel(
        _node_accum_body,
        out_type=jax.ShapeDtypeStruct((NC, NPAD, D), jnp.float32),
        mesh=_sc_mesh(),
        scratch_types=[
            pltpu.VMEM((CPT, CHUNK), jnp.int32),
            pltpu.VMEM((CPT, CHUNK), jnp.int32),
            pltpu.VMEM((CHUNK, D), jnp.float32),
            pltpu.VMEM_SHARED((NPAD, D), jnp.float32),
            pltpu.SemaphoreType.DMA,
        ],
    )
    xv = kc(xe, e_idx, v_scatter, z_node)

    scale = jnp.reshape(1.0 + eps[0], (1, 1))
    rows_blk = 1000
    out = pl.pallas_call(
        _project_body,
        grid=(N_NODES // rows_blk,),
        in_specs=[
            pl.BlockSpec((rows_blk, D), lambda i: (i, 0)),
            pl.BlockSpec((NC, rows_blk, D), lambda i: (0, i, 0)),
            pl.BlockSpec((D, D), lambda i: (0, 0)),
            pl.BlockSpec(memory_space=pltpu.SMEM),
        ],
        out_specs=pl.BlockSpec((rows_blk, D), lambda i: (i, 0)),
        out_shape=jax.ShapeDtypeStruct((N_NODES, D), jnp.float32),
    )(X, xv, W, scale)
    return out
